# baseline (speedup 1.0000x reference)
import jax
import jax.numpy as jnp
from jax.experimental import pallas as pl

_GRIDS = (16, 32, 64)
_UNITS = (256, 1024, 4096)
_OFFS = (0, 256, 1280)
_UTOT = 5376
_BATCH_BLK = 1024
_CW = 256
_RB = 128


def _som_body(x_ref, x2_ref, w_ref, w2_ref,
              bmu1_ref, bmu2_ref, bmu3_ref, qe1_ref, qe2_ref, qe3_ref):
    x = x_ref[...]
    x2 = x2_ref[...]
    bmu_refs = (bmu1_ref, bmu2_ref, bmu3_ref)
    qe_refs = (qe1_ref, qe2_ref, qe3_ref)
    lane = jax.lax.broadcasted_iota(
        jnp.int32, (_BATCH_BLK, _CW), 1).astype(jnp.float32)
    for off, u, g, bmu_ref, qe_ref in zip(_OFFS, _UNITS, _GRIDS, bmu_refs, qe_refs):
        bestv = None
        bestf = None
        for c in range(u // _CW):
            base = off + c * _CW
            w_c = w_ref[base:base + _CW, :]
            p = jax.lax.dot_general(x, w_c, (((1,), (1,)), ((), ())),
                                    preferred_element_type=jnp.float32)
            d2 = (x2 + p) + w2_ref[:, base:base + _CW]
            if bestv is None:
                bestv, bestf = d2, lane
            else:
                lt = d2 < bestv
                bestv = jnp.minimum(d2, bestv)
                bestf = jnp.where(lt, lane + jnp.float32(c * _CW), bestf)
        m = jnp.min(bestv, axis=1, keepdims=True)
        idxf = jnp.min(jnp.where(bestv == m, bestf, jnp.float32(u)),
                       axis=1, keepdims=True)
        bmu_ref[...] = idxf.astype(jnp.int32)
        qe_ref[...] = m


def kernel(x, W1, W2, W3):
    batch = x.shape[0]
    wcat = -2.0 * jnp.concatenate([W1, W2, W3], axis=0)
    w2 = jnp.concatenate([jnp.sum(W1 * W1, axis=1),
                          jnp.sum(W2 * W2, axis=1),
                          jnp.sum(W3 * W3, axis=1)])[None, :]
    x2 = jnp.sum(x * x, axis=1, keepdims=True)
    blk = _BATCH_BLK
    grid = (batch // blk,)
    out_types = (
        jax.ShapeDtypeStruct((batch, 1), jnp.int32),
        jax.ShapeDtypeStruct((batch, 1), jnp.int32),
        jax.ShapeDtypeStruct((batch, 1), jnp.int32),
        jax.ShapeDtypeStruct((batch, 1), jnp.float32),
        jax.ShapeDtypeStruct((batch, 1), jnp.float32),
        jax.ShapeDtypeStruct((batch, 1), jnp.float32),
    )
    bmu_spec = pl.BlockSpec((blk, 1), lambda i: (i, 0))
    qe_spec = pl.BlockSpec((blk, 1), lambda i: (i, 0))
    outs = pl.pallas_call(
        _som_body,
        grid=grid,
        in_specs=[
            pl.BlockSpec((blk, 64), lambda i: (i, 0)),
            pl.BlockSpec((blk, 1), lambda i: (i, 0)),
            pl.BlockSpec((_UTOT, 64), lambda i: (0, 0)),
            pl.BlockSpec((1, _UTOT), lambda i: (0, 0)),
        ],
        out_specs=[bmu_spec, bmu_spec, bmu_spec, qe_spec, qe_spec, qe_spec],
        out_shape=out_types,
    )(x, x2, wcat, w2)
    idx1, idx2, idx3, m1, m2, m3 = outs
    bmus = tuple(
        jnp.stack([idx[:, 0] // g, idx[:, 0] % g], axis=-1)
        for idx, g in zip((idx1, idx2, idx3), _GRIDS))
    qes = tuple(jnp.sqrt(jnp.maximum(m.reshape(batch), 1e-12))
                for m in (m1, m2, m3))
    return bmus + qes


# speedup vs baseline: 1.2043x; 1.2043x over previous
import jax
import jax.numpy as jnp
from jax.experimental import pallas as pl

_GRIDS = (16, 32, 64)
_UNITS = (256, 1024, 4096)
_OFFS = (0, 256, 1280)
_UTOT = 5376
_BATCH_BLK = 2048
_CW = 256
_RB = 128


def _som_body(x_ref, x2_ref, w_ref, w2_ref,
              bmu1_ref, bmu2_ref, bmu3_ref, qe1_ref, qe2_ref, qe3_ref):
    x = x_ref[...]
    x2 = x2_ref[...]
    bmu_refs = (bmu1_ref, bmu2_ref, bmu3_ref)
    qe_refs = (qe1_ref, qe2_ref, qe3_ref)
    lane = jax.lax.broadcasted_iota(
        jnp.int32, (_BATCH_BLK, _CW), 1).astype(jnp.float32)
    for off, u, g, bmu_ref, qe_ref in zip(_OFFS, _UNITS, _GRIDS, bmu_refs, qe_refs):
        bestv = None
        bestf = None
        for c in range(u // _CW):
            base = off + c * _CW
            w_c = w_ref[base:base + _CW, :]
            p = jax.lax.dot_general(x, w_c, (((1,), (1,)), ((), ())),
                                    preferred_element_type=jnp.float32)
            d2 = (x2 + p) + w2_ref[:, base:base + _CW]
            if bestv is None:
                bestv, bestf = d2, lane
            else:
                lt = d2 < bestv
                bestv = jnp.minimum(d2, bestv)
                bestf = jnp.where(lt, lane + jnp.float32(c * _CW), bestf)
        m = jnp.min(bestv, axis=1, keepdims=True)
        idxf = jnp.min(jnp.where(bestv == m, bestf, jnp.float32(u)),
                       axis=1, keepdims=True)
        idx = idxf.astype(jnp.int32)
        bmu_ref[:, 0:1] = idx // g
        bmu_ref[:, 1:2] = idx % g
        qe_ref[...] = jnp.sqrt(jnp.maximum(m, 1e-12))


def kernel(x, W1, W2, W3):
    batch = x.shape[0]
    wcat = -2.0 * jnp.concatenate([W1, W2, W3], axis=0)
    w2 = jnp.concatenate([jnp.sum(W1 * W1, axis=1),
                          jnp.sum(W2 * W2, axis=1),
                          jnp.sum(W3 * W3, axis=1)])[None, :]
    x2 = jnp.sum(x * x, axis=1, keepdims=True)
    blk = _BATCH_BLK
    grid = (batch // blk,)
    out_types = (
        jax.ShapeDtypeStruct((batch, 2), jnp.int32),
        jax.ShapeDtypeStruct((batch, 2), jnp.int32),
        jax.ShapeDtypeStruct((batch, 2), jnp.int32),
        jax.ShapeDtypeStruct((batch, 1), jnp.float32),
        jax.ShapeDtypeStruct((batch, 1), jnp.float32),
        jax.ShapeDtypeStruct((batch, 1), jnp.float32),
    )
    bmu_spec = pl.BlockSpec((blk, 2), lambda i: (i, 0))
    qe_spec = pl.BlockSpec((blk, 1), lambda i: (i, 0))
    outs = pl.pallas_call(
        _som_body,
        grid=grid,
        in_specs=[
            pl.BlockSpec((blk, 64), lambda i: (i, 0)),
            pl.BlockSpec((blk, 1), lambda i: (i, 0)),
            pl.BlockSpec((_UTOT, 64), lambda i: (0, 0)),
            pl.BlockSpec((1, _UTOT), lambda i: (0, 0)),
        ],
        out_specs=[bmu_spec, bmu_spec, bmu_spec, qe_spec, qe_spec, qe_spec],
        out_shape=out_types,
    )(x, x2, wcat, w2)
    bmu1, bmu2, bmu3, qe1, qe2, qe3 = outs
    return (bmu1, bmu2, bmu3,
            qe1.reshape(batch), qe2.reshape(batch), qe3.reshape(batch))


# speedup vs baseline: 1.2160x; 1.0097x over previous
import jax
import jax.numpy as jnp
from jax.experimental import pallas as pl

_GRIDS = (16, 32, 64)
_UNITS = (256, 1024, 4096)
_OFFS = (0, 256, 1280)
_UTOT = 5376
_BATCH_BLK = 4096
_CW = 256
_RB = 128


def _som_body(x_ref, x2_ref, w_ref, w2_ref,
              bmu1_ref, bmu2_ref, bmu3_ref, qe1_ref, qe2_ref, qe3_ref):
    x = x_ref[...]
    x2 = x2_ref[...]
    bmu_refs = (bmu1_ref, bmu2_ref, bmu3_ref)
    qe_refs = (qe1_ref, qe2_ref, qe3_ref)
    lane = jax.lax.broadcasted_iota(
        jnp.int32, (_BATCH_BLK, _CW), 1).astype(jnp.float32)
    for off, u, g, bmu_ref, qe_ref in zip(_OFFS, _UNITS, _GRIDS, bmu_refs, qe_refs):
        bestv = None
        bestf = None
        for c in range(u // _CW):
            base = off + c * _CW
            w_c = w_ref[base:base + _CW, :]
            p = jax.lax.dot_general(x, w_c, (((1,), (1,)), ((), ())),
                                    preferred_element_type=jnp.float32)
            d2 = (x2 + p) + w2_ref[:, base:base + _CW]
            if bestv is None:
                bestv, bestf = d2, lane
            else:
                lt = d2 < bestv
                bestv = jnp.minimum(d2, bestv)
                bestf = jnp.where(lt, lane + jnp.float32(c * _CW), bestf)
        m = jnp.min(bestv, axis=1, keepdims=True)
        idxf = jnp.min(jnp.where(bestv == m, bestf, jnp.float32(u)),
                       axis=1, keepdims=True)
        idx = idxf.astype(jnp.int32)
        bmu_ref[:, 0:1] = idx // g
        bmu_ref[:, 1:2] = idx % g
        qe_ref[...] = jnp.sqrt(jnp.maximum(m, 1e-12))


def kernel(x, W1, W2, W3):
    batch = x.shape[0]
    wcat = -2.0 * jnp.concatenate([W1, W2, W3], axis=0)
    w2 = jnp.concatenate([jnp.sum(W1 * W1, axis=1),
                          jnp.sum(W2 * W2, axis=1),
                          jnp.sum(W3 * W3, axis=1)])[None, :]
    x2 = jnp.sum(x * x, axis=1, keepdims=True)
    blk = _BATCH_BLK
    grid = (batch // blk,)
    out_types = (
        jax.ShapeDtypeStruct((batch, 2), jnp.int32),
        jax.ShapeDtypeStruct((batch, 2), jnp.int32),
        jax.ShapeDtypeStruct((batch, 2), jnp.int32),
        jax.ShapeDtypeStruct((batch, 1), jnp.float32),
        jax.ShapeDtypeStruct((batch, 1), jnp.float32),
        jax.ShapeDtypeStruct((batch, 1), jnp.float32),
    )
    bmu_spec = pl.BlockSpec((blk, 2), lambda i: (i, 0))
    qe_spec = pl.BlockSpec((blk, 1), lambda i: (i, 0))
    outs = pl.pallas_call(
        _som_body,
        grid=grid,
        in_specs=[
            pl.BlockSpec((blk, 64), lambda i: (i, 0)),
            pl.BlockSpec((blk, 1), lambda i: (i, 0)),
            pl.BlockSpec((_UTOT, 64), lambda i: (0, 0)),
            pl.BlockSpec((1, _UTOT), lambda i: (0, 0)),
        ],
        out_specs=[bmu_spec, bmu_spec, bmu_spec, qe_spec, qe_spec, qe_spec],
        out_shape=out_types,
    )(x, x2, wcat, w2)
    bmu1, bmu2, bmu3, qe1, qe2, qe3 = outs
    return (bmu1, bmu2, bmu3,
            qe1.reshape(batch), qe2.reshape(batch), qe3.reshape(batch))
